# Initial kernel scaffold; baseline (speedup 1.0000x reference)
import jax
import jax.numpy as jnp
from jax.experimental import pallas as pl


def kernel(x, edge_index, cycle_index, batch, W_emb, b_emb, conv_W1, conv_b1, conv_W2, conv_b2, bn_g, bn_b, lin_W, lin_b):
    raise NotImplementedError("write your pallas kernel here")


# speedup vs baseline: 3.6593x; 3.6593x over previous
import functools

import jax
import jax.numpy as jnp
from jax import lax
from jax.experimental import pallas as pl
from jax.experimental.pallas import tpu as pltpu
from jax.experimental.pallas import tpu_sc as plsc

N = 10000
E = 320000
IN = 128
H = 256
OUT = 128
L = 3
G = 128

HALF = H // 2
NS = 16
NC = 2
CHUNK = 128
EPAD = ((E + NS * CHUNK - 1) // (NS * CHUNK)) * (NS * CHUNK)
EDGES_PER_TILE = EPAD // NS
CHUNKS_PER_TILE = EDGES_PER_TILE // CHUNK
ACC_ROWS = 10240
ROWS_PER_TILE = 640

BM = 400
NBLK = N // BM


def _sc_agg_body(cur_t, src_idx, dst_idx, out, acc, sidx, didx, rows, sem):
    c = lax.axis_index("c")
    s = lax.axis_index("s")
    row0 = s * ROWS_PER_TILE

    @pl.when(s < NS - 1)
    def _():
        pltpu.sync_copy(cur_t.at[pl.ds(c * N + row0, ROWS_PER_TILE)],
                        acc.at[pl.ds(row0, ROWS_PER_TILE)])

    @pl.when(s == NS - 1)
    def _():
        last = N - (NS - 1) * ROWS_PER_TILE
        pltpu.sync_copy(cur_t.at[pl.ds(c * N + (NS - 1) * ROWS_PER_TILE, last)],
                        acc.at[pl.ds((NS - 1) * ROWS_PER_TILE, last)])

    plsc.subcore_barrier()

    base0 = s * EDGES_PER_TILE
    off = c * N

    def chunk(j, carry):
        b = base0 + j * CHUNK
        pltpu.sync_copy(src_idx.at[pl.ds(b, CHUNK)], sidx)
        pltpu.sync_copy(dst_idx.at[pl.ds(b, CHUNK)], didx)
        for k in range(CHUNK // 16):
            sidx[pl.ds(k * 16, 16)] = sidx[pl.ds(k * 16, 16)] + off
        pltpu.async_copy(cur_t.at[sidx], rows, sem).wait()
        pltpu.sync_copy(rows, acc.at[didx], add=True)
        return carry

    lax.fori_loop(0, CHUNKS_PER_TILE, chunk, 0)

    plsc.subcore_barrier()

    @pl.when(s < NS - 1)
    def _():
        pltpu.sync_copy(acc.at[pl.ds(row0, ROWS_PER_TILE)],
                        out.at[pl.ds(c * N + row0, ROWS_PER_TILE)])

    @pl.when(s == NS - 1)
    def _():
        last = N - (NS - 1) * ROWS_PER_TILE
        pltpu.sync_copy(acc.at[pl.ds((NS - 1) * ROWS_PER_TILE, last)],
                        out.at[pl.ds(c * N + (NS - 1) * ROWS_PER_TILE, last)])


@functools.cache
def _sc_agg_kernel():
    return pl.kernel(
        _sc_agg_body,
        out_type=jax.ShapeDtypeStruct((NC * N, HALF), jnp.float32),
        mesh=plsc.VectorSubcoreMesh(core_axis_name="c", subcore_axis_name="s",
                                    num_cores=NC, num_subcores=NS),
        scratch_types=[
            pltpu.VMEM_SHARED((ACC_ROWS, HALF), jnp.float32),
            pltpu.VMEM((CHUNK,), jnp.int32),
            pltpu.VMEM((CHUNK,), jnp.int32),
            pltpu.VMEM((CHUNK, HALF), jnp.float32),
            pltpu.SemaphoreType.DMA,
        ],
    )


def _sc_agg(cur_t, src_p, dst_p):
    return _sc_agg_kernel()(cur_t, src_p, dst_p)


def _emb_body(x_ref, w_ref, b_ref, o_ref):
    o_ref[...] = jnp.dot(x_ref[...], w_ref[...],
                         preferred_element_type=jnp.float32) + b_ref[...]


def _mlp_body(m0_ref, m1_ref, w1_ref, b1_ref, w2_ref, b2_ref, m2_ref, st_ref):
    m = jnp.concatenate([m0_ref[...], m1_ref[...]], axis=1)
    h = jnp.maximum(jnp.dot(m, w1_ref[...],
                            preferred_element_type=jnp.float32) + b1_ref[...], 0.0)
    m2 = jnp.dot(h, w2_ref[...], preferred_element_type=jnp.float32) + b2_ref[...]
    m2_ref[...] = m2
    st = jnp.concatenate([jnp.sum(m2, 0, keepdims=True),
                          jnp.sum(m2 * m2, 0, keepdims=True)], axis=0)
    i = pl.program_id(0)

    @pl.when(i == 0)
    def _():
        st_ref[...] = st

    @pl.when(i > 0)
    def _():
        st_ref[...] = st_ref[...] + st


def _bn_pool_body(first, m2_ref, cur_ref, st_ref, g_ref, bb_ref, lw_ref,
                  pin_ref, batch_ref, cur_out_ref, pool_ref):
    i = pl.program_id(0)
    st = st_ref[...]
    mean = st[0:1, :] * (1.0 / N)
    var = st[1:2, :] * (1.0 / N) - mean * mean
    inv = lax.rsqrt(var + 1e-5)
    xa = jnp.maximum((m2_ref[...] - mean) * (inv * g_ref[...]) + bb_ref[...], 0.0)
    curn = xa + cur_ref[...]
    cur_out_ref[...] = curn
    z = jnp.dot(curn, lw_ref[...], preferred_element_type=jnp.float32)
    bt = batch_ref[0, 0, :]
    oh = (bt[:, None] == lax.broadcasted_iota(jnp.int32, (BM, G), 1)
          ).astype(jnp.float32)
    contrib = lax.dot_general(oh, z, (((0,), (0,)), ((), ())),
                              preferred_element_type=jnp.float32)

    @pl.when(i == 0)
    def _():
        if first:
            base = jnp.broadcast_to(jnp.sum(pin_ref[...], 0, keepdims=True),
                                    (G, OUT))
        else:
            base = pin_ref[...]
        pool_ref[...] = base + contrib

    @pl.when(i > 0)
    def _():
        pool_ref[...] = pool_ref[...] + contrib


def _emb_call(x, w, b):
    return pl.pallas_call(
        _emb_body,
        grid=(NBLK,),
        in_specs=[
            pl.BlockSpec((BM, IN), lambda i: (i, 0)),
            pl.BlockSpec((IN, H), lambda i: (0, 0)),
            pl.BlockSpec((1, H), lambda i: (0, 0)),
        ],
        out_specs=pl.BlockSpec((BM, H), lambda i: (i, 0)),
        out_shape=jax.ShapeDtypeStruct((N, H), jnp.float32),
    )(x, w, b)


def _mlp_call(m_split, w1, b1, w2, b2):
    return pl.pallas_call(
        _mlp_body,
        grid=(NBLK,),
        in_specs=[
            pl.BlockSpec((BM, HALF), lambda i: (i, 0)),
            pl.BlockSpec((BM, HALF), lambda i: (i + NBLK, 0)),
            pl.BlockSpec((H, H), lambda i: (0, 0)),
            pl.BlockSpec((1, H), lambda i: (0, 0)),
            pl.BlockSpec((H, H), lambda i: (0, 0)),
            pl.BlockSpec((1, H), lambda i: (0, 0)),
        ],
        out_specs=[
            pl.BlockSpec((BM, H), lambda i: (i, 0)),
            pl.BlockSpec((2, H), lambda i: (0, 0)),
        ],
        out_shape=[
            jax.ShapeDtypeStruct((N, H), jnp.float32),
            jax.ShapeDtypeStruct((2, H), jnp.float32),
        ],
    )(m_split, m_split, w1, b1, w2, b2)


def _bn_pool_call(first, m2, cur, st, g, bb, lw, pin, batch_r):
    pin_spec = (pl.BlockSpec((L, OUT), lambda i: (0, 0)) if first
                else pl.BlockSpec((G, OUT), lambda i: (0, 0)))
    return pl.pallas_call(
        functools.partial(_bn_pool_body, first),
        grid=(NBLK,),
        in_specs=[
            pl.BlockSpec((BM, H), lambda i: (i, 0)),
            pl.BlockSpec((BM, H), lambda i: (i, 0)),
            pl.BlockSpec((2, H), lambda i: (0, 0)),
            pl.BlockSpec((1, H), lambda i: (0, 0)),
            pl.BlockSpec((1, H), lambda i: (0, 0)),
            pl.BlockSpec((H, OUT), lambda i: (0, 0)),
            pin_spec,
            pl.BlockSpec((1, 1, BM), lambda i: (i, 0, 0)),
        ],
        out_specs=[
            pl.BlockSpec((BM, H), lambda i: (i, 0)),
            pl.BlockSpec((G, OUT), lambda i: (0, 0)),
        ],
        out_shape=[
            jax.ShapeDtypeStruct((N, H), jnp.float32),
            jax.ShapeDtypeStruct((G, OUT), jnp.float32),
        ],
    )(m2, cur, st, g, bb, lw, pin, batch_r)


def _split_layout(cur):
    return cur.reshape(N, 2, HALF).transpose(1, 0, 2).reshape(2 * N, HALF)


def kernel(x, edge_index, cycle_index, batch, W_emb, b_emb, conv_W1, conv_b1,
           conv_W2, conv_b2, bn_g, bn_b, lin_W, lin_b):
    src = edge_index[0].astype(jnp.int32)
    dst = edge_index[1].astype(jnp.int32)
    pad = EPAD - E
    src_p = jnp.concatenate([src, jnp.zeros((pad,), jnp.int32)])
    dst_p = jnp.concatenate([dst, jnp.full((pad,), N, jnp.int32)])
    batch_r = batch.astype(jnp.int32).reshape(NBLK, 1, BM)

    cur = _emb_call(x, W_emb, b_emb.reshape(1, H))

    pool = lin_b
    for i in range(L):
        cur_t = _split_layout(cur)
        m_split = _sc_agg(cur_t, src_p, dst_p)
        m2, st = _mlp_call(m_split, conv_W1[i], conv_b1[i].reshape(1, H),
                           conv_W2[i], conv_b2[i].reshape(1, H))
        cur, pool = _bn_pool_call(i == 0, m2, cur, st,
                                  bn_g[i].reshape(1, H), bn_b[i].reshape(1, H),
                                  lin_W[i], pool, batch_r)
    return pool


# speedup vs baseline: 4.1562x; 1.1358x over previous
import functools

import jax
import jax.numpy as jnp
from jax import lax
from jax.experimental import pallas as pl
from jax.experimental.pallas import tpu as pltpu
from jax.experimental.pallas import tpu_sc as plsc

N = 10000
E = 320000
IN = 128
H = 256
OUT = 128
L = 3
G = 128

HALF = H // 2
NS = 16
NC = 2
CHUNK = 128
RING = 2
EPAD = ((E + NS * CHUNK * RING - 1) // (NS * CHUNK * RING)) * (NS * CHUNK * RING)
EDGES_PER_TILE = EPAD // NS
CHUNKS_PER_TILE = EDGES_PER_TILE // CHUNK
NGROUPS = CHUNKS_PER_TILE // RING
ACC_ROWS = 10048
ROWS_PER_TILE = 640

BM = 400
NBLK = N // BM


def _sc_agg_body(cur_t, src_idx, dst_idx, out, acc, sidx, didx, rows,
                 gsem, ssem):
    c = lax.axis_index("c")
    s = lax.axis_index("s")
    row0 = s * ROWS_PER_TILE

    @pl.when(s < NS - 1)
    def _():
        pltpu.sync_copy(cur_t.at[pl.ds(c * N + row0, ROWS_PER_TILE)],
                        acc.at[pl.ds(row0, ROWS_PER_TILE)])

    @pl.when(s == NS - 1)
    def _():
        last = N - (NS - 1) * ROWS_PER_TILE
        pltpu.sync_copy(cur_t.at[pl.ds(c * N + (NS - 1) * ROWS_PER_TILE, last)],
                        acc.at[pl.ds((NS - 1) * ROWS_PER_TILE, last)])

    plsc.subcore_barrier()

    base0 = s * EDGES_PER_TILE
    off = c * N

    def load_idx(g, r):
        b = base0 + (g * RING + r) * CHUNK
        pltpu.sync_copy(src_idx.at[pl.ds(b, CHUNK)], sidx.at[r])
        pltpu.sync_copy(dst_idx.at[pl.ds(b, CHUNK)], didx.at[r])
        for k in range(CHUNK // 16):
            sidx[r, pl.ds(k * 16, 16)] = sidx[r, pl.ds(k * 16, 16)] + off

    for r in range(RING):
        load_idx(0, r)
        pltpu.async_copy(cur_t.at[sidx.at[r]], rows.at[r], gsem.at[r])

    def group(g, carry):
        for r in range(RING):
            pltpu.make_async_copy(cur_t.at[sidx.at[r]], rows.at[r],
                                  gsem.at[r]).wait()
            pltpu.async_copy(rows.at[r], acc.at[didx.at[r]], ssem.at[r],
                             add=True)
        for r in range(RING):
            pltpu.make_async_copy(rows.at[r], acc.at[didx.at[r]],
                                  ssem.at[r]).wait()

            @pl.when(g < NGROUPS - 1)
            def _():
                load_idx(g + 1, r)
                pltpu.async_copy(cur_t.at[sidx.at[r]], rows.at[r], gsem.at[r])
        return carry

    lax.fori_loop(0, NGROUPS, group, 0)

    plsc.subcore_barrier()

    @pl.when(s < NS - 1)
    def _():
        pltpu.sync_copy(acc.at[pl.ds(row0, ROWS_PER_TILE)],
                        out.at[pl.ds(c * N + row0, ROWS_PER_TILE)])

    @pl.when(s == NS - 1)
    def _():
        last = N - (NS - 1) * ROWS_PER_TILE
        pltpu.sync_copy(acc.at[pl.ds((NS - 1) * ROWS_PER_TILE, last)],
                        out.at[pl.ds(c * N + (NS - 1) * ROWS_PER_TILE, last)])


@functools.cache
def _sc_agg_kernel():
    return pl.kernel(
        _sc_agg_body,
        out_type=jax.ShapeDtypeStruct((NC * N, HALF), jnp.float32),
        mesh=plsc.VectorSubcoreMesh(core_axis_name="c", subcore_axis_name="s",
                                    num_cores=NC, num_subcores=NS),
        scratch_types=[
            pltpu.VMEM_SHARED((ACC_ROWS, HALF), jnp.float32),
            pltpu.VMEM((RING, CHUNK), jnp.int32),
            pltpu.VMEM((RING, CHUNK), jnp.int32),
            pltpu.VMEM((RING, CHUNK, HALF), jnp.float32),
            pltpu.SemaphoreType.DMA((RING,)),
            pltpu.SemaphoreType.DMA((RING,)),
        ],
    )


def _sc_agg(cur_t, src_p, dst_p):
    return _sc_agg_kernel()(cur_t, src_p, dst_p)


def _emb_body(x_ref, w_ref, b_ref, o_ref):
    o_ref[...] = jnp.dot(x_ref[...], w_ref[...],
                         preferred_element_type=jnp.float32) + b_ref[...]


def _mlp_body(m0_ref, m1_ref, w1_ref, b1_ref, w2_ref, b2_ref, m2_ref, st_ref):
    m = jnp.concatenate([m0_ref[...], m1_ref[...]], axis=1)
    h = jnp.maximum(jnp.dot(m, w1_ref[...],
                            preferred_element_type=jnp.float32) + b1_ref[...], 0.0)
    m2 = jnp.dot(h, w2_ref[...], preferred_element_type=jnp.float32) + b2_ref[...]
    m2_ref[...] = m2
    st = jnp.concatenate([jnp.sum(m2, 0, keepdims=True),
                          jnp.sum(m2 * m2, 0, keepdims=True)], axis=0)
    i = pl.program_id(0)

    @pl.when(i == 0)
    def _():
        st_ref[...] = st

    @pl.when(i > 0)
    def _():
        st_ref[...] = st_ref[...] + st


def _bn_pool_body(first, m2_ref, cur_ref, st_ref, g_ref, bb_ref, lw_ref,
                  pin_ref, batch_ref, cur_out_ref, pool_ref):
    i = pl.program_id(0)
    st = st_ref[...]
    mean = st[0:1, :] * (1.0 / N)
    var = st[1:2, :] * (1.0 / N) - mean * mean
    inv = lax.rsqrt(var + 1e-5)
    xa = jnp.maximum((m2_ref[...] - mean) * (inv * g_ref[...]) + bb_ref[...], 0.0)
    curn = xa + cur_ref[...]
    cur_out_ref[...] = curn
    z = jnp.dot(curn, lw_ref[...], preferred_element_type=jnp.float32)
    bt = batch_ref[0, 0, :]
    oh = (bt[:, None] == lax.broadcasted_iota(jnp.int32, (BM, G), 1)
          ).astype(jnp.float32)
    contrib = lax.dot_general(oh, z, (((0,), (0,)), ((), ())),
                              preferred_element_type=jnp.float32)

    @pl.when(i == 0)
    def _():
        if first:
            base = jnp.broadcast_to(jnp.sum(pin_ref[...], 0, keepdims=True),
                                    (G, OUT))
        else:
            base = pin_ref[...]
        pool_ref[...] = base + contrib

    @pl.when(i > 0)
    def _():
        pool_ref[...] = pool_ref[...] + contrib


def _emb_call(x, w, b):
    return pl.pallas_call(
        _emb_body,
        grid=(NBLK,),
        in_specs=[
            pl.BlockSpec((BM, IN), lambda i: (i, 0)),
            pl.BlockSpec((IN, H), lambda i: (0, 0)),
            pl.BlockSpec((1, H), lambda i: (0, 0)),
        ],
        out_specs=pl.BlockSpec((BM, H), lambda i: (i, 0)),
        out_shape=jax.ShapeDtypeStruct((N, H), jnp.float32),
    )(x, w, b)


def _mlp_call(m_split, w1, b1, w2, b2):
    return pl.pallas_call(
        _mlp_body,
        grid=(NBLK,),
        in_specs=[
            pl.BlockSpec((BM, HALF), lambda i: (i, 0)),
            pl.BlockSpec((BM, HALF), lambda i: (i + NBLK, 0)),
            pl.BlockSpec((H, H), lambda i: (0, 0)),
            pl.BlockSpec((1, H), lambda i: (0, 0)),
            pl.BlockSpec((H, H), lambda i: (0, 0)),
            pl.BlockSpec((1, H), lambda i: (0, 0)),
        ],
        out_specs=[
            pl.BlockSpec((BM, H), lambda i: (i, 0)),
            pl.BlockSpec((2, H), lambda i: (0, 0)),
        ],
        out_shape=[
            jax.ShapeDtypeStruct((N, H), jnp.float32),
            jax.ShapeDtypeStruct((2, H), jnp.float32),
        ],
    )(m_split, m_split, w1, b1, w2, b2)


def _bn_pool_call(first, m2, cur, st, g, bb, lw, pin, batch_r):
    pin_spec = (pl.BlockSpec((L, OUT), lambda i: (0, 0)) if first
                else pl.BlockSpec((G, OUT), lambda i: (0, 0)))
    return pl.pallas_call(
        functools.partial(_bn_pool_body, first),
        grid=(NBLK,),
        in_specs=[
            pl.BlockSpec((BM, H), lambda i: (i, 0)),
            pl.BlockSpec((BM, H), lambda i: (i, 0)),
            pl.BlockSpec((2, H), lambda i: (0, 0)),
            pl.BlockSpec((1, H), lambda i: (0, 0)),
            pl.BlockSpec((1, H), lambda i: (0, 0)),
            pl.BlockSpec((H, OUT), lambda i: (0, 0)),
            pin_spec,
            pl.BlockSpec((1, 1, BM), lambda i: (i, 0, 0)),
        ],
        out_specs=[
            pl.BlockSpec((BM, H), lambda i: (i, 0)),
            pl.BlockSpec((G, OUT), lambda i: (0, 0)),
        ],
        out_shape=[
            jax.ShapeDtypeStruct((N, H), jnp.float32),
            jax.ShapeDtypeStruct((G, OUT), jnp.float32),
        ],
    )(m2, cur, st, g, bb, lw, pin, batch_r)


def _split_layout(cur):
    return cur.reshape(N, 2, HALF).transpose(1, 0, 2).reshape(2 * N, HALF)


def kernel(x, edge_index, cycle_index, batch, W_emb, b_emb, conv_W1, conv_b1,
           conv_W2, conv_b2, bn_g, bn_b, lin_W, lin_b):
    src = edge_index[0].astype(jnp.int32)
    dst = edge_index[1].astype(jnp.int32)
    pad = EPAD - E
    src_p = jnp.concatenate([src, jnp.zeros((pad,), jnp.int32)])
    dst_p = jnp.concatenate([dst, jnp.full((pad,), N, jnp.int32)])
    batch_r = batch.astype(jnp.int32).reshape(NBLK, 1, BM)

    cur = _emb_call(x, W_emb, b_emb.reshape(1, H))

    pool = lin_b
    for i in range(L):
        cur_t = _split_layout(cur)
        m_split = _sc_agg(cur_t, src_p, dst_p)
        m2, st = _mlp_call(m_split, conv_W1[i], conv_b1[i].reshape(1, H),
                           conv_W2[i], conv_b2[i].reshape(1, H))
        cur, pool = _bn_pool_call(i == 0, m2, cur, st,
                                  bn_g[i].reshape(1, H), bn_b[i].reshape(1, H),
                                  lin_W[i], pool, batch_r)
    return pool
